# Initial kernel scaffold; baseline (speedup 1.0000x reference)
import jax
import jax.numpy as jnp
from jax.experimental import pallas as pl


def kernel(values, kinds, Bmat, kind_emb, W1, b1, W2, b2):
    raise NotImplementedError("write your pallas kernel here")


# speedup vs baseline: 4.0814x; 4.0814x over previous
import functools
import math

import jax
import jax.numpy as jnp
from jax import lax
from jax.experimental import pallas as pl
from jax.experimental.pallas import tpu as pltpu
from jax.experimental.pallas import tpu_sc as plsc

_M = 256


def _mlp_body(e_ref, vals_ref, bcol_ref, w1_ref, b1_ref, w2_ref, b2_ref, out_ref):
    v = vals_ref[0]
    yt = bcol_ref[0] * v
    fft = jnp.concatenate([jnp.sin(yt), jnp.cos(yt)], axis=0)
    h = lax.dot_general(fft, w1_ref[0], (((0,), (0,)), ((), ())),
                        preferred_element_type=jnp.float32)
    h = h + b1_ref[0]
    h = 0.5 * h * (1.0 + lax.erf(h * (1.0 / math.sqrt(2.0))))
    out_ref[...] = jnp.dot(h, w2_ref[0],
                           preferred_element_type=jnp.float32) + b2_ref[0]


def kernel(values, kinds, Bmat, kind_emb, W1, b1, W2, b2):
    N, S, _ = values.shape
    T = N * S
    K, d = kind_emb.shape
    B = Bmat.shape[1]
    Bp = ((B + 31) // 32) * 32
    M = _M
    TP = T // M + K - 1
    Tpad = TP * M

    f32 = jnp.float32

    kflat = kinds.reshape(T).astype(jnp.int32)
    onehot = (kflat[:, None] == jnp.arange(K, dtype=jnp.int32)[None, :])
    csum = jnp.cumsum(onehot.astype(jnp.int32), axis=0)
    counts = csum[-1]
    rank = jnp.take_along_axis(csum, kflat[:, None], axis=1)[:, 0] - 1
    ntiles = (counts + M - 1) // M
    tiles_cum = jnp.cumsum(ntiles)
    tile_start = tiles_cum - ntiles
    pos = (tile_start[kflat] * M + rank).astype(jnp.int32)
    expert_of_tile = jnp.clip(
        jnp.searchsorted(tiles_cum, jnp.arange(TP, dtype=jnp.int32),
                         side="right"), 0, K - 1).astype(jnp.int32)
    gsrc = jnp.zeros((Tpad,), jnp.int32).at[pos].set(
        jnp.arange(T, dtype=jnp.int32))

    zpad = jnp.zeros((K, Bp - B, d), f32)
    W1p = jnp.concatenate([W1[:, :B], zpad, W1[:, B:], zpad], axis=1)
    b1r = b1.reshape(K, 1, d)
    b2r = (b2 + kind_emb).reshape(K, 1, d)
    bcol = jnp.pad((2.0 * math.pi) * Bmat[0], (0, Bp - B)).reshape(1, Bp, 1)

    vals_flat = values.reshape(T)

    mesh = plsc.VectorSubcoreMesh(core_axis_name="c", subcore_axis_name="s")
    NC, NS = 2, 16
    NW = NC * NS

    spw = Tpad // NW

    @functools.partial(
        pl.kernel, mesh=mesh,
        out_type=jax.ShapeDtypeStruct((Tpad,), f32),
        scratch_types=[
            pltpu.VMEM((spw,), jnp.int32),
            pltpu.VMEM((spw,), f32),
            pltpu.SemaphoreType.DMA,
        ],
    )
    def permute_vals(vals_hbm, gsrc_hbm, out_hbm, idx_v, buf_v, sem):
        wid = lax.axis_index("s") * NC + lax.axis_index("c")
        base = wid * spw
        pltpu.sync_copy(gsrc_hbm.at[pl.ds(base, spw)], idx_v)
        pltpu.async_copy(vals_hbm.at[idx_v], buf_v, sem).wait()
        pltpu.sync_copy(buf_v, out_hbm.at[pl.ds(base, spw)])

    vals_sorted = permute_vals(vals_flat, gsrc)

    grid_spec = pltpu.PrefetchScalarGridSpec(
        num_scalar_prefetch=1,
        grid=(TP,),
        in_specs=[
            pl.BlockSpec((1, 1, M), lambda i, e: (i, 0, 0)),
            pl.BlockSpec((1, Bp, 1), lambda i, e: (0, 0, 0)),
            pl.BlockSpec((1, 2 * Bp, d), lambda i, e: (e[i], 0, 0)),
            pl.BlockSpec((1, 1, d), lambda i, e: (e[i], 0, 0)),
            pl.BlockSpec((1, d, d), lambda i, e: (e[i], 0, 0)),
            pl.BlockSpec((1, 1, d), lambda i, e: (e[i], 0, 0)),
        ],
        out_specs=pl.BlockSpec((M, d), lambda i, e: (i, 0)),
    )
    out_sorted = pl.pallas_call(
        _mlp_body,
        grid_spec=grid_spec,
        out_shape=jax.ShapeDtypeStruct((Tpad, d), f32),
        compiler_params=pltpu.CompilerParams(
            dimension_semantics=("arbitrary",)),
    )(expert_of_tile, vals_sorted.reshape(TP, 1, M), bcol, W1p, b1r, W2, b2r)

    C = 32
    rows_per_w = T // NW

    @functools.partial(
        pl.kernel, mesh=mesh,
        out_type=jax.ShapeDtypeStruct((T, d), f32),
        scratch_types=[
            pltpu.VMEM((C,), jnp.int32),
            pltpu.VMEM((C, d), f32),
            pltpu.SemaphoreType.DMA,
        ],
    )
    def gather_rows(table_hbm, pos_hbm, out_hbm, idx_v, rows_v, sem):
        wid = lax.axis_index("s") * NC + lax.axis_index("c")
        base = wid * rows_per_w

        def body(c, carry):
            b = base + c * C
            pltpu.sync_copy(pos_hbm.at[pl.ds(b, C)], idx_v)
            pltpu.async_copy(table_hbm.at[idx_v], rows_v, sem).wait()
            pltpu.sync_copy(rows_v, out_hbm.at[pl.ds(b, C)])
            return carry

        lax.fori_loop(0, rows_per_w // C, body, 0)

    out = gather_rows(out_sorted, pos)
    return out.reshape(N, S, d)
